# baseline (speedup 1.0000x reference)
import functools

import jax
import jax.numpy as jnp
import numpy as np
from jax import lax
from jax.experimental import pallas as pl
from jax.experimental.pallas import tpu as pltpu
from jax.experimental.pallas import tpu_sc as plsc

B = 16
P = 64
M = 32
N_DIM = 256
H = 64
K_IT = 10
E = 512


def _threefry2x32(k0, k1, x0, x1):
    rot = (13, 15, 26, 6, 17, 29, 16, 24)
    k0 = np.uint32(k0)
    k1 = np.uint32(k1)
    ks = (k0, k1, np.uint32(k0 ^ k1 ^ np.uint32(0x1BD11BDA)))
    x0 = (x0 + ks[0]).astype(np.uint32)
    x1 = (x1 + ks[1]).astype(np.uint32)
    for i in range(5):
        for r in rot[(i % 2) * 4:(i % 2) * 4 + 4]:
            x0 = (x0 + x1).astype(np.uint32)
            x1 = ((x1 << np.uint32(r)) | (x1 >> np.uint32(32 - r))).astype(np.uint32)
            x1 = (x1 ^ x0).astype(np.uint32)
        x0 = (x0 + ks[(i + 1) % 3]).astype(np.uint32)
        x1 = (x1 + ks[(i + 2) % 3] + np.uint32(i + 1)).astype(np.uint32)
    return x0, x1


def _erfinv64(x):
    x = x.astype(np.float64)
    w = -np.log1p(-x * x)
    p_lo = np.full_like(w, 2.81022636e-08)
    wl = w - 2.5
    for c in (3.43273939e-07, -3.5233877e-06, -4.39150654e-06, 0.00021858087,
              -0.00125372503, -0.00417768164, 0.246640727, 1.50140941):
        p_lo = c + p_lo * wl
    ws = np.sqrt(np.maximum(w, 5.0)) - 3.0
    p_hi = np.full_like(w, -0.000200214257)
    for c in (0.000100950558, 0.00134934322, -0.00367342844, 0.00573950773,
              -0.0076224613, 0.00943887047, 1.00167406, 2.83297682):
        p_hi = c + p_hi * ws
    return np.where(w < 5.0, p_lo, p_hi) * x


def _init_state():
    n = B * P * N_DIM
    with np.errstate(over="ignore"):
        s1, s2 = _threefry2x32(0, 1, np.zeros(3, np.uint32),
                               np.arange(3, dtype=np.uint32))
        out = []
        for i in range(3):
            b1, b2 = _threefry2x32(s1[i], s2[i], np.zeros(n, np.uint32),
                                   np.arange(n, dtype=np.uint32))
            bits = (b1 ^ b2).astype(np.uint32)
            f = ((bits >> np.uint32(9)) | np.uint32(0x3F800000)).view(np.float32)
            f = f - np.float32(1.0)
            lo = np.float32(np.nextafter(np.float32(-1.0), np.float32(0.0)))
            u = np.maximum(lo, (f * (np.float32(1.0) - lo) + lo).astype(np.float32))
            v = (np.sqrt(2.0) * _erfinv64(u)).astype(np.float32)
            v = v.reshape(B, P, N_DIM)
            out.append(np.transpose(v, (1, 0, 2)) * np.float32(0.01))
    return out


_Y0, _U0, _D0 = _init_state()


def _leaky(x):
    return jnp.where(x >= 0, x, 0.01 * x)


def _sc_edge_body(edge_hbm, c_hbm, src_v, dst_v, cnt_v):
    cid = lax.axis_index("c")
    sid = lax.axis_index("s")

    @pl.when(cid == 0)
    def _():
        bb = sid
        pltpu.sync_copy(edge_hbm.at[bb, 0], src_v)
        pltpu.sync_copy(edge_hbm.at[bb, 1], dst_v)
        zeros16 = jnp.zeros((16,), jnp.float32)

        def zbody(i, carry):
            cnt_v[pl.ds(i * 16, 16)] = zeros16
            return carry

        lax.fori_loop(0, P * P // 16, zbody, 0)
        lanes = lax.iota(jnp.int32, 16)
        ones16 = jnp.ones((16,), jnp.float32)
        for ch in range(E // 16):
            s = src_v[pl.ds(ch * 16, 16)]
            d = dst_v[pl.ds(ch * 16, 16)]
            flat = d * P + s
            for l in range(16):
                plsc.addupdate_scatter(cnt_v, [flat], ones16,
                                       mask=lanes == l)
        pltpu.sync_copy(cnt_v, c_hbm.at[bb])


def _sc_edge_counts(edge):
    mesh = plsc.VectorSubcoreMesh(core_axis_name="c", subcore_axis_name="s")
    fn = functools.partial(
        pl.kernel,
        mesh=mesh,
        out_type=jax.ShapeDtypeStruct((B, P * P), jnp.float32),
        scratch_types=[
            pltpu.VMEM((E,), jnp.int32),
            pltpu.VMEM((E,), jnp.int32),
            pltpu.VMEM((P * P,), jnp.float32),
        ],
        compiler_params=pltpu.CompilerParams(needs_layout_passes=False),
    )(_sc_edge_body)
    return fn(edge)


_KB = 1024
_NB = 1024
_L1_STEPS = (2048 // _KB) * (4096 // _NB)
_L2_STEPS = (4096 // _KB) * (8192 // _NB)
_L3_STEPS = (8192 // _KB) * (16384 // _NB)
_S1 = _L1_STEPS
_S2 = _S1 + _L2_STEPS
_S3 = _S2 + _L3_STEPS
_GRID = _S3 + 1 + K_IT


def _mono_kernel(x0_ref, w1_ref, b1_ref, w2_ref, b2_ref, w3_ref, b3_ref,
                 c_ref, wc1_ref, bc1_ref, wc2_ref, bc2_ref,
                 wf1_ref, bf1_ref, wf2_ref, bf2_ref, mp_ref,
                 a0_ref, bt_ref, y0_ref, u0_ref, d0_ref, o_ref,
                 x1_s, x2_s, x3_s, acc_s, lap_s, sn_s,
                 atb_s, y_s, u_s, d_s, ha_s, ht_s, hr_s, he_s):
    f32 = jnp.float32
    s = pl.program_id(0)

    @pl.when(s < _S1)
    def _layer1():
        k1 = s % (2048 // _KB)
        j1 = s // (2048 // _KB)

        @pl.when(k1 == 0)
        def _():
            acc_s[...] = jnp.zeros_like(acc_s)

        acc_s[...] += jnp.dot(x0_ref[:, pl.ds(k1 * _KB, _KB)], w1_ref[...],
                              preferred_element_type=f32)

        @pl.when(k1 == 2048 // _KB - 1)
        def _():
            x1_s[:, pl.ds(j1 * _NB, _NB)] = _leaky(acc_s[...] + b1_ref[...])

    @pl.when(jnp.logical_and(s >= _S1, s < _S2))
    def _layer2():
        t = s - _S1
        k2 = t % (4096 // _KB)
        j2 = t // (4096 // _KB)

        @pl.when(k2 == 0)
        def _():
            acc_s[...] = jnp.zeros_like(acc_s)

        acc_s[...] += jnp.dot(x1_s[:, pl.ds(k2 * _KB, _KB)], w2_ref[...],
                              preferred_element_type=f32)

        @pl.when(k2 == 4096 // _KB - 1)
        def _():
            x2_s[:, pl.ds(j2 * _NB, _NB)] = _leaky(acc_s[...] + b2_ref[...])

    @pl.when(jnp.logical_and(s >= _S2, s < _S3))
    def _layer3():
        t = s - _S2
        k3 = t % (8192 // _KB)
        j3 = t // (8192 // _KB)

        @pl.when(k3 == 0)
        def _():
            acc_s[...] = jnp.zeros_like(acc_s)

        acc_s[...] += jnp.dot(x2_s[:, pl.ds(k3 * _KB, _KB)], w3_ref[...],
                              preferred_element_type=f32)

        @pl.when(k3 == 8192 // _KB - 1)
        def _():
            x3_s[:, pl.ds(j3 * _NB, _NB)] = acc_s[...] + b3_ref[...]

    @pl.when(s == _S3)
    def _prologue():
        c = jnp.reshape(c_ref[...], (B, P, P))
        ii = lax.broadcasted_iota(jnp.int32, (P, P), 0)
        jj = lax.broadcasted_iota(jnp.int32, (P, P), 1)
        eye = (ii == jj).astype(f32)
        ct = lax.dot_general(c, eye, (((1,), (0,)), ((), ())),
                             preferred_element_type=f32)
        deg_d = jnp.sum(c, axis=2)
        deg_s = jnp.sum(ct, axis=2)
        dinv = lax.rsqrt(deg_d + 1.0)
        adj = dinv[:, :, None] * dinv[:, None, :] * (c + eye[None])
        lap_s[...] = eye[None] * (deg_s + deg_d)[:, :, None] - c - ct
        sn_s[...] = lax.dot_general(eye, deg_s, (((1,), (1,)), ((), ())),
                                    preferred_element_type=f32)

        x = jnp.reshape(x3_s[...], (B, P, 4 * H))
        xw = lax.dot_general(x, wc1_ref[...], (((2,), (0,)), ((), ())),
                             preferred_element_type=f32)
        h = lax.dot_general(adj, xw, (((2,), (1,)), ((0,), (0,))),
                            preferred_element_type=f32)
        h = _leaky(h + bc1_ref[...][None])
        hw = lax.dot_general(h, wc2_ref[...], (((2,), (0,)), ((), ())),
                             preferred_element_type=f32)
        h2 = lax.dot_general(adj, hw, (((2,), (1,)), ((0,), (0,))),
                             preferred_element_type=f32)
        h2 = _leaky(h2 + bc2_ref[...][None])
        pool = jnp.mean(h2, axis=1)
        f = _leaky(jnp.dot(pool, wf1_ref[...],
                           preferred_element_type=f32) + bf1_ref[...])
        g = jnp.dot(f, wf2_ref[...],
                    preferred_element_type=f32) + bf2_ref[...]
        mp = mp_ref[...]

        qq = lax.broadcasted_iota(jnp.int32, (P * 4, P), 0)
        pp = lax.broadcasted_iota(jnp.int32, (P * 4, P), 1)
        refs = (ha_s, ht_s, hr_s, he_s)
        acc = jnp.zeros((B, P * 4), f32)
        for k in range(K_IT):
            acc = acc + g[:, k * P * 4:(k + 1) * P * 4]
            hyp_k = jax.nn.sigmoid(acc) * mp
            for j in range(4):
                sel = (qq == 4 * pp + j).astype(f32)
                refs[j][k] = lax.dot_general(sel, hyp_k,
                                             (((0,), (1,)), ((), ())),
                                             preferred_element_type=f32)

        a0 = a0_ref[...]
        atb_s[...] = lax.dot_general(bt_ref[...], a0,
                                     (((2,), (1,)), ((0,), (0,))),
                                     preferred_element_type=f32)
        y_s[...] = y0_ref[...]
        u_s[...] = u0_ref[...]
        d_s[...] = d0_ref[...]

    @pl.when(s > _S3)
    def _admm_step():
        k = s - _S3 - 1
        a0 = a0_ref[...]
        sn = sn_s[...][:, :, None]
        al = jnp.reshape(ha_s[pl.ds(k, 1)], (P, B))[:, :, None]
        ta = jnp.reshape(ht_s[pl.ds(k, 1)], (P, B))[:, :, None]
        rh = jnp.reshape(hr_s[pl.ds(k, 1)], (P, B))[:, :, None]
        et = jnp.reshape(he_s[pl.ds(k, 1)], (P, B))[:, :, None]
        y = y_s[...]
        ay = lax.dot_general(y, a0, (((2,), (2,)), ((0,), (0,))),
                             preferred_element_type=f32)
        atay = lax.dot_general(ay, a0, (((2,), (1,)), ((0,), (0,))),
                               preferred_element_type=f32)
        grad = (atay - atb_s[...] + jnp.sign(y) * ta
                + u_s[...] * sn + d_s[...] * rh)
        y_next = y - al * grad
        for bb in range(B):
            yb = y_next[:, bb, :]
            db = jnp.dot(lap_s[bb], yb, preferred_element_type=f32)
            d_s[:, bb, :] = db
            o_ref[0, bb] = yb
        u_s[...] = u_s[...] + d_s[...] * et
        y_s[...] = y_next


def _const_map(rank):
    return lambda s: (0,) * rank


def _mono(x0, w1, b1, w2, b2, w3, b3, c4, wc1, bc1, wc2, bc2,
          wf1, bf1, wf2, bf2, mp, a0, bt, y0, u0, d0):
    def w1_map(s):
        t = jnp.clip(s, 0, _L1_STEPS - 1)
        return (t % 2, t // 2)

    def b1_map(s):
        t = jnp.clip(s, 0, _L1_STEPS - 1)
        return (0, t // 2)

    def w2_map(s):
        t = jnp.clip(s - _S1, 0, _L2_STEPS - 1)
        return (t % 4, t // 4)

    def b2_map(s):
        t = jnp.clip(s - _S1, 0, _L2_STEPS - 1)
        return (0, t // 4)

    def w3_map(s):
        t = jnp.clip(s - _S2, 0, _L3_STEPS - 1)
        return (t % 8, t // 8)

    def b3_map(s):
        t = jnp.clip(s - _S2, 0, _L3_STEPS - 1)
        return (0, t // 8)

    def o_map(s):
        return (jnp.clip(s - _S3 - 1, 0, K_IT - 1), 0, 0, 0)

    full = lambda arr: pl.BlockSpec(arr.shape, _const_map(arr.ndim))
    return pl.pallas_call(
        _mono_kernel,
        grid=(_GRID,),
        in_specs=[
            full(x0),
            pl.BlockSpec((_KB, _NB), w1_map),
            pl.BlockSpec((1, _NB), b1_map),
            pl.BlockSpec((_KB, _NB), w2_map),
            pl.BlockSpec((1, _NB), b2_map),
            pl.BlockSpec((_KB, _NB), w3_map),
            pl.BlockSpec((1, _NB), b3_map),
            full(c4), full(wc1), full(bc1), full(wc2), full(bc2),
            full(wf1), full(bf1), full(wf2), full(bf2), full(mp),
            full(a0), full(bt), full(y0), full(u0), full(d0),
        ],
        out_specs=pl.BlockSpec((1, B, P, N_DIM), o_map),
        out_shape=jax.ShapeDtypeStruct((K_IT, B, P, N_DIM), jnp.float32),
        scratch_shapes=[
            pltpu.VMEM((B, 4096), jnp.float32),
            pltpu.VMEM((B, 8192), jnp.float32),
            pltpu.VMEM((B, 16384), jnp.float32),
            pltpu.VMEM((B, _NB), jnp.float32),
            pltpu.VMEM((B, P, P), jnp.float32),
            pltpu.VMEM((P, B), jnp.float32),
            pltpu.VMEM((P, B, N_DIM), jnp.float32),
            pltpu.VMEM((P, B, N_DIM), jnp.float32),
            pltpu.VMEM((P, B, N_DIM), jnp.float32),
            pltpu.VMEM((P, B, N_DIM), jnp.float32),
            pltpu.VMEM((K_IT, P, B), jnp.float32),
            pltpu.VMEM((K_IT, P, B), jnp.float32),
            pltpu.VMEM((K_IT, P, B), jnp.float32),
            pltpu.VMEM((K_IT, P, B), jnp.float32),
        ],
        compiler_params=pltpu.CompilerParams(
            dimension_semantics=("arbitrary",)),
    )(x0, w1, b1, w2, b2, w3, b3, c4, wc1, bc1, wc2, bc2,
      wf1, bf1, wf2, bf2, mp, a0, bt, y0, u0, d0)


def kernel(b, A, W1, b1, W2, b2, W3, b3, Wc1, bc1, Wc2, bc2,
           Wf1, bf1, Wf2, bf2, max_param, edge_index):
    edge = edge_index.astype(jnp.int32)
    c4 = _sc_edge_counts(edge)

    x0 = b.reshape(B, P * M)
    mp = jnp.tile(max_param.reshape(-1), P).reshape(1, P * 4)
    a0 = A[0]
    bt = jnp.transpose(b[..., 0], (1, 0, 2))

    ys = _mono(x0, W1, b1.reshape(1, -1), W2, b2.reshape(1, -1),
               W3, b3.reshape(1, -1), c4, Wc1, bc1.reshape(1, -1),
               Wc2, bc2.reshape(1, -1), Wf1, bf1.reshape(1, -1),
               Wf2, bf2.reshape(1, -1), mp, a0, bt,
               jnp.asarray(_Y0), jnp.asarray(_U0), jnp.asarray(_D0))
    return ys[..., None]


# speedup vs baseline: 1.0528x; 1.0528x over previous
import functools

import jax
import jax.numpy as jnp
import numpy as np
from jax import lax
from jax.experimental import pallas as pl
from jax.experimental.pallas import tpu as pltpu
from jax.experimental.pallas import tpu_sc as plsc

B = 16
P = 64
M = 32
N_DIM = 256
H = 64
K_IT = 10
E = 512


def _threefry2x32(k0, k1, x0, x1):
    rot = (13, 15, 26, 6, 17, 29, 16, 24)
    k0 = np.uint32(k0)
    k1 = np.uint32(k1)
    ks = (k0, k1, np.uint32(k0 ^ k1 ^ np.uint32(0x1BD11BDA)))
    x0 = (x0 + ks[0]).astype(np.uint32)
    x1 = (x1 + ks[1]).astype(np.uint32)
    for i in range(5):
        for r in rot[(i % 2) * 4:(i % 2) * 4 + 4]:
            x0 = (x0 + x1).astype(np.uint32)
            x1 = ((x1 << np.uint32(r)) | (x1 >> np.uint32(32 - r))).astype(np.uint32)
            x1 = (x1 ^ x0).astype(np.uint32)
        x0 = (x0 + ks[(i + 1) % 3]).astype(np.uint32)
        x1 = (x1 + ks[(i + 2) % 3] + np.uint32(i + 1)).astype(np.uint32)
    return x0, x1


def _erfinv64(x):
    x = x.astype(np.float64)
    w = -np.log1p(-x * x)
    p_lo = np.full_like(w, 2.81022636e-08)
    wl = w - 2.5
    for c in (3.43273939e-07, -3.5233877e-06, -4.39150654e-06, 0.00021858087,
              -0.00125372503, -0.00417768164, 0.246640727, 1.50140941):
        p_lo = c + p_lo * wl
    ws = np.sqrt(np.maximum(w, 5.0)) - 3.0
    p_hi = np.full_like(w, -0.000200214257)
    for c in (0.000100950558, 0.00134934322, -0.00367342844, 0.00573950773,
              -0.0076224613, 0.00943887047, 1.00167406, 2.83297682):
        p_hi = c + p_hi * ws
    return np.where(w < 5.0, p_lo, p_hi) * x


def _init_state():
    n = B * P * N_DIM
    with np.errstate(over="ignore"):
        s1, s2 = _threefry2x32(0, 1, np.zeros(3, np.uint32),
                               np.arange(3, dtype=np.uint32))
        out = []
        for i in range(3):
            b1, b2 = _threefry2x32(s1[i], s2[i], np.zeros(n, np.uint32),
                                   np.arange(n, dtype=np.uint32))
            bits = (b1 ^ b2).astype(np.uint32)
            f = ((bits >> np.uint32(9)) | np.uint32(0x3F800000)).view(np.float32)
            f = f - np.float32(1.0)
            lo = np.float32(np.nextafter(np.float32(-1.0), np.float32(0.0)))
            u = np.maximum(lo, (f * (np.float32(1.0) - lo) + lo).astype(np.float32))
            v = (np.sqrt(2.0) * _erfinv64(u)).astype(np.float32)
            v = v.reshape(B, P, N_DIM)
            out.append(np.transpose(v, (1, 0, 2)) * np.float32(0.01))
    return out


_Y0, _U0, _D0 = _init_state()


def _leaky(x):
    return jnp.where(x >= 0, x, 0.01 * x)


def _sc_edge_body(edge_hbm, c_hbm, src_v, dst_v, cnt_v):
    cid = lax.axis_index("c")
    sid = lax.axis_index("s")

    @pl.when(cid == 0)
    def _():
        bb = sid
        pltpu.sync_copy(edge_hbm.at[bb, 0], src_v)
        pltpu.sync_copy(edge_hbm.at[bb, 1], dst_v)
        zeros16 = jnp.zeros((16,), jnp.float32)

        def zbody(i, carry):
            cnt_v[pl.ds(i * 16, 16)] = zeros16
            return carry

        lax.fori_loop(0, P * P // 16, zbody, 0)
        lanes = lax.iota(jnp.int32, 16)
        ones16 = jnp.ones((16,), jnp.float32)
        for ch in range(E // 16):
            s = src_v[pl.ds(ch * 16, 16)]
            d = dst_v[pl.ds(ch * 16, 16)]
            flat = d * P + s
            for l in range(16):
                plsc.addupdate_scatter(cnt_v, [flat], ones16,
                                       mask=lanes == l)
        pltpu.sync_copy(cnt_v, c_hbm.at[bb])


def _sc_edge_counts(edge):
    mesh = plsc.VectorSubcoreMesh(core_axis_name="c", subcore_axis_name="s")
    fn = functools.partial(
        pl.kernel,
        mesh=mesh,
        out_type=jax.ShapeDtypeStruct((B, P * P), jnp.float32),
        scratch_types=[
            pltpu.VMEM((E,), jnp.int32),
            pltpu.VMEM((E,), jnp.int32),
            pltpu.VMEM((P * P,), jnp.float32),
        ],
        compiler_params=pltpu.CompilerParams(needs_layout_passes=False),
    )(_sc_edge_body)
    return fn(edge)


_M1 = 8
_M2 = _M1 + 16
_M3 = _M2 + 32


def _mlp3_kernel(x0_ref, w1_ref, b1_ref, w2_ref, b2_ref, w3_ref, b3_ref,
                 o_ref, x1_s, x2_s, acc1_s, acc2_s):
    f32 = jnp.float32
    s = pl.program_id(0)

    @pl.when(s < _M1)
    def _layer1():
        o = jnp.dot(x0_ref[...], w1_ref[...], preferred_element_type=f32)
        x1_s[:, pl.ds(s * 512, 512)] = _leaky(o + b1_ref[...])

    @pl.when(jnp.logical_and(s >= _M1, s < _M2))
    def _layer2():
        t = s - _M1
        k2 = t % 2
        j2 = t // 2

        @pl.when(k2 == 0)
        def _():
            acc1_s[...] = jnp.zeros_like(acc1_s)

        acc1_s[...] += jnp.dot(x1_s[:, pl.ds(k2 * 2048, 2048)], w2_ref[...],
                               preferred_element_type=f32)

        @pl.when(k2 == 1)
        def _():
            x2_s[:, pl.ds(j2 * 1024, 1024)] = _leaky(acc1_s[...] + b2_ref[...])

    @pl.when(s >= _M2)
    def _layer3():
        t = s - _M2
        k3 = t % 4
        j3 = t // 4

        @pl.when(k3 == 0)
        def _():
            acc2_s[...] = jnp.zeros_like(acc2_s)

        acc2_s[...] += jnp.dot(x2_s[:, pl.ds(k3 * 2048, 2048)], w3_ref[...],
                               preferred_element_type=f32)

        @pl.when(k3 == 3)
        def _():
            o_ref[...] = acc2_s[...] + b3_ref[...]


def _mlp3(x0, w1, b1, w2, b2, w3, b3):
    def w1_map(s):
        return (0, jnp.clip(s, 0, _M1 - 1))

    def w2_map(s):
        t = jnp.clip(s - _M1, 0, 15)
        return (t % 2, t // 2)

    def b2_map(s):
        t = jnp.clip(s - _M1, 0, 15)
        return (0, t // 2)

    def w3_map(s):
        t = jnp.clip(s - _M2, 0, 31)
        return (t % 4, t // 4)

    def b3_map(s):
        t = jnp.clip(s - _M2, 0, 31)
        return (0, t // 4)

    return pl.pallas_call(
        _mlp3_kernel,
        grid=(_M3,),
        in_specs=[
            pl.BlockSpec((B, 2048), lambda s: (0, 0)),
            pl.BlockSpec((2048, 512), w1_map),
            pl.BlockSpec((1, 512), w1_map),
            pl.BlockSpec((2048, 1024), w2_map),
            pl.BlockSpec((1, 1024), b2_map),
            pl.BlockSpec((2048, 2048), w3_map),
            pl.BlockSpec((1, 2048), b3_map),
        ],
        out_specs=pl.BlockSpec((B, 2048), b3_map),
        out_shape=jax.ShapeDtypeStruct((B, 16384), jnp.float32),
        scratch_shapes=[
            pltpu.VMEM((B, 4096), jnp.float32),
            pltpu.VMEM((B, 8192), jnp.float32),
            pltpu.VMEM((B, 1024), jnp.float32),
            pltpu.VMEM((B, 2048), jnp.float32),
        ],
        compiler_params=pltpu.CompilerParams(
            dimension_semantics=("arbitrary",)),
    )(x0, w1, b1.reshape(1, -1), w2, b2.reshape(1, -1),
      w3, b3.reshape(1, -1))


def _mega_kernel(c_ref, x_ref, wc1_ref, bc1_ref, wc2_ref, bc2_ref,
                 wf1_ref, bf1_ref, wf2_ref, bf2_ref, mp_ref,
                 a0_ref, bt_ref, y0_ref, u0_ref, d0_ref, o_ref,
                 atb_ref, y_ref, u_ref, d_ref,
                 ha_ref, ht_ref, hr_ref, he_ref):
    f32 = jnp.float32
    c = jnp.reshape(c_ref[...], (B, P, P))
    ii = lax.broadcasted_iota(jnp.int32, (P, P), 0)
    jj = lax.broadcasted_iota(jnp.int32, (P, P), 1)
    eye = (ii == jj).astype(f32)
    ct = lax.dot_general(c, eye, (((1,), (0,)), ((), ())),
                         preferred_element_type=f32)
    deg_d = jnp.sum(c, axis=2)
    deg_s = jnp.sum(ct, axis=2)
    dinv = lax.rsqrt(deg_d + 1.0)
    adj = dinv[:, :, None] * dinv[:, None, :] * (c + eye[None])
    lap = eye[None] * (deg_s + deg_d)[:, :, None] - c - ct
    sn = lax.dot_general(eye, deg_s, (((1,), (1,)), ((), ())),
                         preferred_element_type=f32)[:, :, None]

    x = jnp.reshape(x_ref[...], (B, P, 4 * H))
    xw = lax.dot_general(x, wc1_ref[...], (((2,), (0,)), ((), ())),
                         preferred_element_type=f32)
    h = lax.dot_general(adj, xw, (((2,), (1,)), ((0,), (0,))),
                        preferred_element_type=f32)
    h = _leaky(h + bc1_ref[...][None])
    hw = lax.dot_general(h, wc2_ref[...], (((2,), (0,)), ((), ())),
                         preferred_element_type=f32)
    h2 = lax.dot_general(adj, hw, (((2,), (1,)), ((0,), (0,))),
                         preferred_element_type=f32)
    h2 = _leaky(h2 + bc2_ref[...][None])
    pool = jnp.mean(h2, axis=1)
    f = _leaky(jnp.dot(pool, wf1_ref[...],
                       preferred_element_type=f32) + bf1_ref[...])
    g = jnp.dot(f, wf2_ref[...],
                preferred_element_type=f32) + bf2_ref[...]
    mp = mp_ref[...]

    qq = lax.broadcasted_iota(jnp.int32, (P * 4, P), 0)
    pp = lax.broadcasted_iota(jnp.int32, (P * 4, P), 1)
    refs = (ha_ref, ht_ref, hr_ref, he_ref)
    acc = jnp.zeros((B, P * 4), f32)
    for k in range(K_IT):
        acc = acc + g[:, k * P * 4:(k + 1) * P * 4]
        hyp_k = jax.nn.sigmoid(acc) * mp
        for j in range(4):
            sel = (qq == 4 * pp + j).astype(f32)
            refs[j][k] = lax.dot_general(sel, hyp_k, (((0,), (1,)), ((), ())),
                                         preferred_element_type=f32)

    a0 = a0_ref[...]
    atb_ref[...] = lax.dot_general(bt_ref[...], a0,
                                   (((2,), (1,)), ((0,), (0,))),
                                   preferred_element_type=f32)
    y_ref[...] = y0_ref[...]
    u_ref[...] = u0_ref[...]
    d_ref[...] = d0_ref[...]

    def step(k, _):
        al = jnp.reshape(ha_ref[pl.ds(k, 1)], (P, B))[:, :, None]
        ta = jnp.reshape(ht_ref[pl.ds(k, 1)], (P, B))[:, :, None]
        rh = jnp.reshape(hr_ref[pl.ds(k, 1)], (P, B))[:, :, None]
        et = jnp.reshape(he_ref[pl.ds(k, 1)], (P, B))[:, :, None]
        y = y_ref[...]
        ay = lax.dot_general(y, a0, (((2,), (2,)), ((0,), (0,))),
                             preferred_element_type=f32)
        atay = lax.dot_general(ay, a0, (((2,), (1,)), ((0,), (0,))),
                               preferred_element_type=f32)
        grad = (atay - atb_ref[...] + jnp.sign(y) * ta
                + u_ref[...] * sn + d_ref[...] * rh)
        y_next = y - al * grad
        for bb in range(B):
            yb = y_next[:, bb, :]
            db = jnp.dot(lap[bb], yb, preferred_element_type=f32)
            d_ref[:, bb, :] = db
            o_ref[pl.ds(k, 1), bb] = yb[None]
        u_ref[...] = u_ref[...] + d_ref[...] * et
        y_ref[...] = y_next
        return 0

    lax.fori_loop(0, K_IT, step, 0)


def _mega(c4, x3, wc1, bc1, wc2, bc2, wf1, bf1, wf2, bf2, mp,
          a0, bt, y0, u0, d0):
    return pl.pallas_call(
        _mega_kernel,
        out_shape=jax.ShapeDtypeStruct((K_IT, B, P, N_DIM), jnp.float32),
        scratch_shapes=[
            pltpu.VMEM((P, B, N_DIM), jnp.float32),
            pltpu.VMEM((P, B, N_DIM), jnp.float32),
            pltpu.VMEM((P, B, N_DIM), jnp.float32),
            pltpu.VMEM((P, B, N_DIM), jnp.float32),
            pltpu.VMEM((K_IT, P, B), jnp.float32),
            pltpu.VMEM((K_IT, P, B), jnp.float32),
            pltpu.VMEM((K_IT, P, B), jnp.float32),
            pltpu.VMEM((K_IT, P, B), jnp.float32),
        ],
    )(c4, x3, wc1, bc1.reshape(1, -1), wc2, bc2.reshape(1, -1),
      wf1, bf1.reshape(1, -1), wf2, bf2.reshape(1, -1), mp,
      a0, bt, y0, u0, d0)


def kernel(b, A, W1, b1, W2, b2, W3, b3, Wc1, bc1, Wc2, bc2,
           Wf1, bf1, Wf2, bf2, max_param, edge_index):
    edge = edge_index.astype(jnp.int32)
    c4 = _sc_edge_counts(edge)

    x0 = b.reshape(B, P * M)
    x3 = _mlp3(x0, W1, b1, W2, b2, W3, b3)

    mp = jnp.tile(max_param.reshape(-1), P).reshape(1, P * 4)
    a0 = A[0]
    bt = jnp.transpose(b[..., 0], (1, 0, 2))

    ys = _mega(c4, x3, Wc1, bc1, Wc2, bc2, Wf1, bf1, Wf2, bf2, mp,
               a0, bt, jnp.asarray(_Y0), jnp.asarray(_U0), jnp.asarray(_D0))
    return ys[..., None]


# speedup vs baseline: 1.0574x; 1.0044x over previous
import functools

import jax
import jax.numpy as jnp
import numpy as np
from jax import lax
from jax.experimental import pallas as pl
from jax.experimental.pallas import tpu as pltpu
from jax.experimental.pallas import tpu_sc as plsc

B = 16
P = 64
M = 32
N_DIM = 256
H = 64
K_IT = 10
E = 512


def _threefry2x32(k0, k1, x0, x1):
    rot = (13, 15, 26, 6, 17, 29, 16, 24)
    k0 = np.uint32(k0)
    k1 = np.uint32(k1)
    ks = (k0, k1, np.uint32(k0 ^ k1 ^ np.uint32(0x1BD11BDA)))
    x0 = (x0 + ks[0]).astype(np.uint32)
    x1 = (x1 + ks[1]).astype(np.uint32)
    for i in range(5):
        for r in rot[(i % 2) * 4:(i % 2) * 4 + 4]:
            x0 = (x0 + x1).astype(np.uint32)
            x1 = ((x1 << np.uint32(r)) | (x1 >> np.uint32(32 - r))).astype(np.uint32)
            x1 = (x1 ^ x0).astype(np.uint32)
        x0 = (x0 + ks[(i + 1) % 3]).astype(np.uint32)
        x1 = (x1 + ks[(i + 2) % 3] + np.uint32(i + 1)).astype(np.uint32)
    return x0, x1


def _erfinv64(x):
    x = x.astype(np.float64)
    w = -np.log1p(-x * x)
    p_lo = np.full_like(w, 2.81022636e-08)
    wl = w - 2.5
    for c in (3.43273939e-07, -3.5233877e-06, -4.39150654e-06, 0.00021858087,
              -0.00125372503, -0.00417768164, 0.246640727, 1.50140941):
        p_lo = c + p_lo * wl
    ws = np.sqrt(np.maximum(w, 5.0)) - 3.0
    p_hi = np.full_like(w, -0.000200214257)
    for c in (0.000100950558, 0.00134934322, -0.00367342844, 0.00573950773,
              -0.0076224613, 0.00943887047, 1.00167406, 2.83297682):
        p_hi = c + p_hi * ws
    return np.where(w < 5.0, p_lo, p_hi) * x


def _init_state():
    n = B * P * N_DIM
    with np.errstate(over="ignore"):
        s1, s2 = _threefry2x32(0, 1, np.zeros(3, np.uint32),
                               np.arange(3, dtype=np.uint32))
        out = []
        for i in range(3):
            b1, b2 = _threefry2x32(s1[i], s2[i], np.zeros(n, np.uint32),
                                   np.arange(n, dtype=np.uint32))
            bits = (b1 ^ b2).astype(np.uint32)
            f = ((bits >> np.uint32(9)) | np.uint32(0x3F800000)).view(np.float32)
            f = f - np.float32(1.0)
            lo = np.float32(np.nextafter(np.float32(-1.0), np.float32(0.0)))
            u = np.maximum(lo, (f * (np.float32(1.0) - lo) + lo).astype(np.float32))
            v = (np.sqrt(2.0) * _erfinv64(u)).astype(np.float32)
            v = v.reshape(B, P, N_DIM)
            out.append(np.transpose(v, (1, 0, 2)) * np.float32(0.01))
    return out


_Y0, _U0, _D0 = _init_state()


def _leaky(x):
    return jnp.where(x >= 0, x, 0.01 * x)


def _sc_edge_body(edge_hbm, c_hbm, src_v, dst_v, cnt_v):
    cid = lax.axis_index("c")
    sid = lax.axis_index("s")

    @pl.when(cid == 0)
    def _():
        bb = sid
        pltpu.sync_copy(edge_hbm.at[bb, 0], src_v)
        pltpu.sync_copy(edge_hbm.at[bb, 1], dst_v)
        zeros16 = jnp.zeros((16,), jnp.float32)

        def zbody(i, carry):
            cnt_v[pl.ds(i * 16, 16)] = zeros16
            return carry

        lax.fori_loop(0, P * P // 16, zbody, 0)
        lanes = lax.iota(jnp.int32, 16)
        ones16 = jnp.ones((16,), jnp.float32)
        for ch in range(E // 16):
            s = src_v[pl.ds(ch * 16, 16)]
            d = dst_v[pl.ds(ch * 16, 16)]
            flat = d * P + s
            for l in range(16):
                plsc.addupdate_scatter(cnt_v, [flat], ones16,
                                       mask=lanes == l)
        pltpu.sync_copy(cnt_v, c_hbm.at[bb])


def _sc_edge_counts(edge):
    mesh = plsc.VectorSubcoreMesh(core_axis_name="c", subcore_axis_name="s")
    fn = functools.partial(
        pl.kernel,
        mesh=mesh,
        out_type=jax.ShapeDtypeStruct((B, P * P), jnp.float32),
        scratch_types=[
            pltpu.VMEM((E,), jnp.int32),
            pltpu.VMEM((E,), jnp.int32),
            pltpu.VMEM((P * P,), jnp.float32),
        ],
        compiler_params=pltpu.CompilerParams(needs_layout_passes=False),
    )(_sc_edge_body)
    return fn(edge)


_M1 = 8
_M2 = _M1 + 16
_M3 = _M2 + 32


def _mlp3_kernel(x0_ref, w1_ref, b1_ref, w2_ref, b2_ref, w3_ref, b3_ref,
                 o_ref, x1_s, x2_s, acc1_s, acc2_s):
    f32 = jnp.float32
    s = pl.program_id(0)

    @pl.when(s < _M1)
    def _layer1():
        o = jnp.dot(x0_ref[...], w1_ref[...], preferred_element_type=f32)
        x1_s[:, pl.ds(s * 512, 512)] = _leaky(o + b1_ref[...])

    @pl.when(jnp.logical_and(s >= _M1, s < _M2))
    def _layer2():
        t = s - _M1
        k2 = t % 2
        j2 = t // 2

        @pl.when(k2 == 0)
        def _():
            acc1_s[...] = jnp.zeros_like(acc1_s)

        acc1_s[...] += jnp.dot(x1_s[:, pl.ds(k2 * 2048, 2048)], w2_ref[...],
                               preferred_element_type=f32)

        @pl.when(k2 == 1)
        def _():
            x2_s[:, pl.ds(j2 * 1024, 1024)] = _leaky(acc1_s[...] + b2_ref[...])

    @pl.when(s >= _M2)
    def _layer3():
        t = s - _M2
        k3 = t % 4
        j3 = t // 4

        @pl.when(k3 == 0)
        def _():
            acc2_s[...] = jnp.zeros_like(acc2_s)

        acc2_s[...] += jnp.dot(x2_s[:, pl.ds(k3 * 2048, 2048)], w3_ref[...],
                               preferred_element_type=f32)

        @pl.when(k3 == 3)
        def _():
            o_ref[...] = acc2_s[...] + b3_ref[...]


def _mlp3(x0, w1, b1, w2, b2, w3, b3):
    def w1_map(s):
        return (0, jnp.clip(s, 0, _M1 - 1))

    def w2_map(s):
        t = jnp.clip(s - _M1, 0, 15)
        return (t % 2, t // 2)

    def b2_map(s):
        t = jnp.clip(s - _M1, 0, 15)
        return (0, t // 2)

    def w3_map(s):
        t = jnp.clip(s - _M2, 0, 31)
        return (t % 4, t // 4)

    def b3_map(s):
        t = jnp.clip(s - _M2, 0, 31)
        return (0, t // 4)

    return pl.pallas_call(
        _mlp3_kernel,
        grid=(_M3,),
        in_specs=[
            pl.BlockSpec((B, 2048), lambda s: (0, 0)),
            pl.BlockSpec((2048, 512), w1_map),
            pl.BlockSpec((1, 512), w1_map),
            pl.BlockSpec((2048, 1024), w2_map),
            pl.BlockSpec((1, 1024), b2_map),
            pl.BlockSpec((2048, 2048), w3_map),
            pl.BlockSpec((1, 2048), b3_map),
        ],
        out_specs=pl.BlockSpec((B, 2048), b3_map),
        out_shape=jax.ShapeDtypeStruct((B, 16384), jnp.float32),
        scratch_shapes=[
            pltpu.VMEM((B, 4096), jnp.float32),
            pltpu.VMEM((B, 8192), jnp.float32),
            pltpu.VMEM((B, 1024), jnp.float32),
            pltpu.VMEM((B, 2048), jnp.float32),
        ],
        compiler_params=pltpu.CompilerParams(
            dimension_semantics=("arbitrary",)),
    )(x0, w1, b1.reshape(1, -1), w2, b2.reshape(1, -1),
      w3, b3.reshape(1, -1))


def _mega_kernel(c_ref, x_ref, wc1_ref, bc1_ref, wc2_ref, bc2_ref,
                 wf1_ref, bf1_ref, wf2_ref, bf2_ref, mp_ref,
                 a0_ref, bt_ref, y0_ref, u0_ref, d0_ref, o_ref,
                 atb_ref, y_ref, u_ref, d_ref,
                 ha_ref, ht_ref, hr_ref, he_ref):
    f32 = jnp.float32
    c = jnp.reshape(c_ref[...], (B, P, P))
    ii = lax.broadcasted_iota(jnp.int32, (P, P), 0)
    jj = lax.broadcasted_iota(jnp.int32, (P, P), 1)
    eye = (ii == jj).astype(f32)
    ct = lax.dot_general(c, eye, (((1,), (0,)), ((), ())),
                         preferred_element_type=f32)
    deg_d = jnp.sum(c, axis=2)
    deg_s = jnp.sum(ct, axis=2)
    dinv = lax.rsqrt(deg_d + 1.0)
    adj = dinv[:, :, None] * dinv[:, None, :] * (c + eye[None])
    lap = eye[None] * (deg_s + deg_d)[:, :, None] - c - ct
    sn = lax.dot_general(eye, deg_s, (((1,), (1,)), ((), ())),
                         preferred_element_type=f32)[:, :, None]

    x = jnp.reshape(x_ref[...], (B, P, 4 * H))
    xw = lax.dot_general(x, wc1_ref[...], (((2,), (0,)), ((), ())),
                         preferred_element_type=f32)
    h = lax.dot_general(adj, xw, (((2,), (1,)), ((0,), (0,))),
                        preferred_element_type=f32)
    h = _leaky(h + bc1_ref[...][None])
    hw = lax.dot_general(h, wc2_ref[...], (((2,), (0,)), ((), ())),
                         preferred_element_type=f32)
    h2 = lax.dot_general(adj, hw, (((2,), (1,)), ((0,), (0,))),
                         preferred_element_type=f32)
    h2 = _leaky(h2 + bc2_ref[...][None])
    pool = jnp.mean(h2, axis=1)
    f = _leaky(jnp.dot(pool, wf1_ref[...],
                       preferred_element_type=f32) + bf1_ref[...])
    g = jnp.dot(f, wf2_ref[...],
                preferred_element_type=f32) + bf2_ref[...]
    tq = lax.broadcasted_iota(jnp.int32, (4, P * 4), 0)
    tp = lax.broadcasted_iota(jnp.int32, (4, P * 4), 1)
    tilemat = (tp % 4 == tq).astype(f32)
    mp = jnp.dot(mp_ref[...], tilemat, preferred_element_type=f32)

    qq = lax.broadcasted_iota(jnp.int32, (P * 4, P), 0)
    pp = lax.broadcasted_iota(jnp.int32, (P * 4, P), 1)
    refs = (ha_ref, ht_ref, hr_ref, he_ref)
    acc = jnp.zeros((B, P * 4), f32)
    for k in range(K_IT):
        acc = acc + g[:, k * P * 4:(k + 1) * P * 4]
        hyp_k = jax.nn.sigmoid(acc) * mp
        for j in range(4):
            sel = (qq == 4 * pp + j).astype(f32)
            refs[j][k] = lax.dot_general(sel, hyp_k, (((0,), (1,)), ((), ())),
                                         preferred_element_type=f32)

    a0 = a0_ref[...]
    atb_ref[...] = lax.dot_general(bt_ref[...], a0,
                                   (((2,), (1,)), ((0,), (0,))),
                                   preferred_element_type=f32)
    y_ref[...] = y0_ref[...]
    u_ref[...] = u0_ref[...]
    d_ref[...] = d0_ref[...]

    def step(k, _):
        al = jnp.reshape(ha_ref[pl.ds(k, 1)], (P, B))[:, :, None]
        ta = jnp.reshape(ht_ref[pl.ds(k, 1)], (P, B))[:, :, None]
        rh = jnp.reshape(hr_ref[pl.ds(k, 1)], (P, B))[:, :, None]
        et = jnp.reshape(he_ref[pl.ds(k, 1)], (P, B))[:, :, None]
        y = y_ref[...]
        ay = lax.dot_general(y, a0, (((2,), (2,)), ((0,), (0,))),
                             preferred_element_type=f32)
        atay = lax.dot_general(ay, a0, (((2,), (1,)), ((0,), (0,))),
                               preferred_element_type=f32)
        grad = (atay - atb_ref[...] + jnp.sign(y) * ta
                + u_ref[...] * sn + d_ref[...] * rh)
        y_next = y - al * grad
        for bb in range(B):
            yb = y_next[:, bb, :]
            db = jnp.dot(lap[bb], yb, preferred_element_type=f32)
            d_ref[:, bb, :] = db
            o_ref[pl.ds(k, 1), bb] = yb[None]
        u_ref[...] = u_ref[...] + d_ref[...] * et
        y_ref[...] = y_next
        return 0

    lax.fori_loop(0, K_IT, step, 0)


def _mega(c4, x3, wc1, bc1, wc2, bc2, wf1, bf1, wf2, bf2, mp,
          a0, bt, y0, u0, d0):
    return pl.pallas_call(
        _mega_kernel,
        out_shape=jax.ShapeDtypeStruct((K_IT, B, P, N_DIM), jnp.float32),
        scratch_shapes=[
            pltpu.VMEM((P, B, N_DIM), jnp.float32),
            pltpu.VMEM((P, B, N_DIM), jnp.float32),
            pltpu.VMEM((P, B, N_DIM), jnp.float32),
            pltpu.VMEM((P, B, N_DIM), jnp.float32),
            pltpu.VMEM((K_IT, P, B), jnp.float32),
            pltpu.VMEM((K_IT, P, B), jnp.float32),
            pltpu.VMEM((K_IT, P, B), jnp.float32),
            pltpu.VMEM((K_IT, P, B), jnp.float32),
        ],
    )(c4, x3, wc1, bc1.reshape(1, -1), wc2, bc2.reshape(1, -1),
      wf1, bf1.reshape(1, -1), wf2, bf2.reshape(1, -1), mp,
      a0, bt, y0, u0, d0)


def kernel(b, A, W1, b1, W2, b2, W3, b3, Wc1, bc1, Wc2, bc2,
           Wf1, bf1, Wf2, bf2, max_param, edge_index):
    edge = edge_index.astype(jnp.int32)
    c4 = _sc_edge_counts(edge)

    x0 = b.reshape(B, P * M)
    x3 = _mlp3(x0, W1, b1, W2, b2, W3, b3)

    mp = max_param.reshape(1, 4)
    a0 = A[0]
    bt = jnp.transpose(b[..., 0], (1, 0, 2))

    ys = _mega(c4, x3, Wc1, bc1, Wc2, bc2, Wf1, bf1, Wf2, bf2, mp,
               a0, bt, jnp.asarray(_Y0), jnp.asarray(_U0), jnp.asarray(_D0))
    return ys[..., None]


# speedup vs baseline: 1.0745x; 1.0161x over previous
import functools

import jax
import jax.numpy as jnp
import numpy as np
from jax import lax
from jax.experimental import pallas as pl
from jax.experimental.pallas import tpu as pltpu
from jax.experimental.pallas import tpu_sc as plsc

B = 16
P = 64
M = 32
N_DIM = 256
H = 64
K_IT = 10
E = 512


def _threefry2x32(k0, k1, x0, x1):
    rot = (13, 15, 26, 6, 17, 29, 16, 24)
    k0 = np.uint32(k0)
    k1 = np.uint32(k1)
    ks = (k0, k1, np.uint32(k0 ^ k1 ^ np.uint32(0x1BD11BDA)))
    x0 = (x0 + ks[0]).astype(np.uint32)
    x1 = (x1 + ks[1]).astype(np.uint32)
    for i in range(5):
        for r in rot[(i % 2) * 4:(i % 2) * 4 + 4]:
            x0 = (x0 + x1).astype(np.uint32)
            x1 = ((x1 << np.uint32(r)) | (x1 >> np.uint32(32 - r))).astype(np.uint32)
            x1 = (x1 ^ x0).astype(np.uint32)
        x0 = (x0 + ks[(i + 1) % 3]).astype(np.uint32)
        x1 = (x1 + ks[(i + 2) % 3] + np.uint32(i + 1)).astype(np.uint32)
    return x0, x1


def _erfinv64(x):
    x = x.astype(np.float64)
    w = -np.log1p(-x * x)
    p_lo = np.full_like(w, 2.81022636e-08)
    wl = w - 2.5
    for c in (3.43273939e-07, -3.5233877e-06, -4.39150654e-06, 0.00021858087,
              -0.00125372503, -0.00417768164, 0.246640727, 1.50140941):
        p_lo = c + p_lo * wl
    ws = np.sqrt(np.maximum(w, 5.0)) - 3.0
    p_hi = np.full_like(w, -0.000200214257)
    for c in (0.000100950558, 0.00134934322, -0.00367342844, 0.00573950773,
              -0.0076224613, 0.00943887047, 1.00167406, 2.83297682):
        p_hi = c + p_hi * ws
    return np.where(w < 5.0, p_lo, p_hi) * x


def _init_state():
    n = B * P * N_DIM
    with np.errstate(over="ignore"):
        s1, s2 = _threefry2x32(0, 1, np.zeros(3, np.uint32),
                               np.arange(3, dtype=np.uint32))
        out = []
        for i in range(3):
            b1, b2 = _threefry2x32(s1[i], s2[i], np.zeros(n, np.uint32),
                                   np.arange(n, dtype=np.uint32))
            bits = (b1 ^ b2).astype(np.uint32)
            f = ((bits >> np.uint32(9)) | np.uint32(0x3F800000)).view(np.float32)
            f = f - np.float32(1.0)
            lo = np.float32(np.nextafter(np.float32(-1.0), np.float32(0.0)))
            u = np.maximum(lo, (f * (np.float32(1.0) - lo) + lo).astype(np.float32))
            v = (np.sqrt(2.0) * _erfinv64(u)).astype(np.float32)
            v = v.reshape(B, P, N_DIM)
            out.append(np.transpose(v, (1, 0, 2)) * np.float32(0.01))
    return out


_Y0, _U0, _D0 = _init_state()


def _leaky(x):
    return jnp.where(x >= 0, x, 0.01 * x)


def _sc_edge_body(edge_hbm, c_hbm, src_v, dst_v, cnt_v):
    cid = lax.axis_index("c")
    sid = lax.axis_index("s")

    @pl.when(cid == 0)
    def _():
        bb = sid
        pltpu.sync_copy(edge_hbm.at[bb, 0], src_v)
        pltpu.sync_copy(edge_hbm.at[bb, 1], dst_v)
        zeros16 = jnp.zeros((16,), jnp.float32)

        def zbody(i, carry):
            cnt_v[pl.ds(i * 16, 16)] = zeros16
            return carry

        lax.fori_loop(0, P * P // 16, zbody, 0)
        lanes = lax.iota(jnp.int32, 16)
        ones16 = jnp.ones((16,), jnp.float32)
        for ch in range(E // 16):
            s = src_v[pl.ds(ch * 16, 16)]
            d = dst_v[pl.ds(ch * 16, 16)]
            flat = d * P + s
            for l in range(16):
                plsc.addupdate_scatter(cnt_v, [flat], ones16,
                                       mask=lanes == l)
        pltpu.sync_copy(cnt_v, c_hbm.at[bb])


def _sc_edge_counts(edge):
    mesh = plsc.VectorSubcoreMesh(core_axis_name="c", subcore_axis_name="s")
    fn = functools.partial(
        pl.kernel,
        mesh=mesh,
        out_type=jax.ShapeDtypeStruct((B, P * P), jnp.float32),
        scratch_types=[
            pltpu.VMEM((E,), jnp.int32),
            pltpu.VMEM((E,), jnp.int32),
            pltpu.VMEM((P * P,), jnp.float32),
        ],
        compiler_params=pltpu.CompilerParams(needs_layout_passes=False),
    )(_sc_edge_body)
    return fn(edge)


_M1 = 8
_M2 = _M1 + 16
_M3 = _M2 + 32


def _mlp3_kernel(x0_ref, w1_ref, b1_ref, w2_ref, b2_ref, w3_ref, b3_ref,
                 o_ref, x1_s, x2_s, acc1_s, acc2_s):
    f32 = jnp.float32
    s = pl.program_id(0)

    @pl.when(s < _M1)
    def _layer1():
        o = jnp.dot(x0_ref[...], w1_ref[...], preferred_element_type=f32)
        x1_s[:, pl.ds(s * 512, 512)] = _leaky(o + b1_ref[...])

    @pl.when(jnp.logical_and(s >= _M1, s < _M2))
    def _layer2():
        t = s - _M1
        k2 = t % 2
        j2 = t // 2

        @pl.when(k2 == 0)
        def _():
            acc1_s[...] = jnp.zeros_like(acc1_s)

        acc1_s[...] += jnp.dot(x1_s[:, pl.ds(k2 * 2048, 2048)], w2_ref[...],
                               preferred_element_type=f32)

        @pl.when(k2 == 1)
        def _():
            x2_s[:, pl.ds(j2 * 1024, 1024)] = _leaky(acc1_s[...] + b2_ref[...])

    @pl.when(s >= _M2)
    def _layer3():
        t = s - _M2
        k3 = t % 4
        j3 = t // 4

        @pl.when(k3 == 0)
        def _():
            acc2_s[...] = jnp.zeros_like(acc2_s)

        acc2_s[...] += jnp.dot(x2_s[:, pl.ds(k3 * 2048, 2048)], w3_ref[...],
                               preferred_element_type=f32)

        @pl.when(k3 == 3)
        def _():
            o_ref[...] = acc2_s[...] + b3_ref[...]


def _mlp3(x0, w1, b1, w2, b2, w3, b3):
    def w1_map(s):
        return (0, jnp.clip(s, 0, _M1 - 1))

    def w2_map(s):
        t = jnp.clip(s - _M1, 0, 15)
        return (t % 2, t // 2)

    def b2_map(s):
        t = jnp.clip(s - _M1, 0, 15)
        return (0, t // 2)

    def w3_map(s):
        t = jnp.clip(s - _M2, 0, 31)
        return (t % 4, t // 4)

    def b3_map(s):
        t = jnp.clip(s - _M2, 0, 31)
        return (0, t // 4)

    return pl.pallas_call(
        _mlp3_kernel,
        grid=(_M3,),
        in_specs=[
            pl.BlockSpec((B, 2048), lambda s: (0, 0)),
            pl.BlockSpec((2048, 512), w1_map),
            pl.BlockSpec((1, 512), w1_map),
            pl.BlockSpec((2048, 1024), w2_map),
            pl.BlockSpec((1, 1024), b2_map),
            pl.BlockSpec((2048, 2048), w3_map),
            pl.BlockSpec((1, 2048), b3_map),
        ],
        out_specs=pl.BlockSpec((B, 2048), b3_map),
        out_shape=jax.ShapeDtypeStruct((B, 16384), jnp.float32),
        scratch_shapes=[
            pltpu.VMEM((B, 4096), jnp.float32),
            pltpu.VMEM((B, 8192), jnp.float32),
            pltpu.VMEM((B, 1024), jnp.float32),
            pltpu.VMEM((B, 2048), jnp.float32),
        ],
        compiler_params=pltpu.CompilerParams(
            dimension_semantics=("arbitrary",)),
    )(x0, w1, b1.reshape(1, -1), w2, b2.reshape(1, -1),
      w3, b3.reshape(1, -1))


def _mega_kernel(c_ref, x_ref, wc1_ref, bc1_ref, wc2_ref, bc2_ref,
                 wf1_ref, bf1_ref, wf2_ref, bf2_ref, mp_ref,
                 a0_ref, bt_ref, y0_ref, u0_ref, d0_ref, o_ref,
                 atb_ref, y_ref, u_ref, d_ref, lap_s, sn_s,
                 ha_ref, ht_ref, hr_ref, he_ref):
    f32 = jnp.float32
    step_id = pl.program_id(0)

    @pl.when(step_id == 0)
    def _prologue():
        _mega_prologue(c_ref, x_ref, wc1_ref, bc1_ref, wc2_ref, bc2_ref,
                       wf1_ref, bf1_ref, wf2_ref, bf2_ref, mp_ref,
                       a0_ref, bt_ref, y0_ref, u0_ref, d0_ref,
                       atb_ref, y_ref, u_ref, d_ref, lap_s, sn_s,
                       ha_ref, ht_ref, hr_ref, he_ref)

    @pl.when(step_id > 0)
    def _admm_step():
        k = step_id - 1
        a0 = a0_ref[...]
        sn = sn_s[...][:, :, None]
        al = jnp.reshape(ha_ref[pl.ds(k, 1)], (P, B))[:, :, None]
        ta = jnp.reshape(ht_ref[pl.ds(k, 1)], (P, B))[:, :, None]
        rh = jnp.reshape(hr_ref[pl.ds(k, 1)], (P, B))[:, :, None]
        et = jnp.reshape(he_ref[pl.ds(k, 1)], (P, B))[:, :, None]
        y = y_ref[...]
        ay = lax.dot_general(y, a0, (((2,), (2,)), ((0,), (0,))),
                             preferred_element_type=f32)
        atay = lax.dot_general(ay, a0, (((2,), (1,)), ((0,), (0,))),
                               preferred_element_type=f32)
        grad = (atay - atb_ref[...] + jnp.sign(y) * ta
                + u_ref[...] * sn + d_ref[...] * rh)
        y_next = y - al * grad
        for bb in range(B):
            yb = y_next[:, bb, :]
            db = jnp.dot(lap_s[bb], yb, preferred_element_type=f32)
            d_ref[:, bb, :] = db
            o_ref[0, bb] = yb
        u_ref[...] = u_ref[...] + d_ref[...] * et
        y_ref[...] = y_next


def _mega_prologue(c_ref, x_ref, wc1_ref, bc1_ref, wc2_ref, bc2_ref,
                   wf1_ref, bf1_ref, wf2_ref, bf2_ref, mp_ref,
                   a0_ref, bt_ref, y0_ref, u0_ref, d0_ref,
                   atb_ref, y_ref, u_ref, d_ref, lap_s, sn_s,
                   ha_ref, ht_ref, hr_ref, he_ref):
    f32 = jnp.float32
    c = jnp.reshape(c_ref[...], (B, P, P))
    ii = lax.broadcasted_iota(jnp.int32, (P, P), 0)
    jj = lax.broadcasted_iota(jnp.int32, (P, P), 1)
    eye = (ii == jj).astype(f32)
    ct = lax.dot_general(c, eye, (((1,), (0,)), ((), ())),
                         preferred_element_type=f32)
    deg_d = jnp.sum(c, axis=2)
    deg_s = jnp.sum(ct, axis=2)
    dinv = lax.rsqrt(deg_d + 1.0)
    adj = dinv[:, :, None] * dinv[:, None, :] * (c + eye[None])
    lap_s[...] = eye[None] * (deg_s + deg_d)[:, :, None] - c - ct
    sn_s[...] = lax.dot_general(eye, deg_s, (((1,), (1,)), ((), ())),
                                preferred_element_type=f32)

    x = jnp.reshape(x_ref[...], (B, P, 4 * H))
    xw = lax.dot_general(x, wc1_ref[...], (((2,), (0,)), ((), ())),
                         preferred_element_type=f32)
    h = lax.dot_general(adj, xw, (((2,), (1,)), ((0,), (0,))),
                        preferred_element_type=f32)
    h = _leaky(h + bc1_ref[...][None])
    hw = lax.dot_general(h, wc2_ref[...], (((2,), (0,)), ((), ())),
                         preferred_element_type=f32)
    h2 = lax.dot_general(adj, hw, (((2,), (1,)), ((0,), (0,))),
                         preferred_element_type=f32)
    h2 = _leaky(h2 + bc2_ref[...][None])
    pool = jnp.mean(h2, axis=1)
    f = _leaky(jnp.dot(pool, wf1_ref[...],
                       preferred_element_type=f32) + bf1_ref[...])
    g = jnp.dot(f, wf2_ref[...],
                preferred_element_type=f32) + bf2_ref[...]
    tq = lax.broadcasted_iota(jnp.int32, (4, P * 4), 0)
    tp = lax.broadcasted_iota(jnp.int32, (4, P * 4), 1)
    tilemat = (tp % 4 == tq).astype(f32)
    mp = jnp.dot(mp_ref[...], tilemat, preferred_element_type=f32)

    qq = lax.broadcasted_iota(jnp.int32, (P * 4, P), 0)
    pp = lax.broadcasted_iota(jnp.int32, (P * 4, P), 1)
    refs = (ha_ref, ht_ref, hr_ref, he_ref)
    acc = jnp.zeros((B, P * 4), f32)
    for k in range(K_IT):
        acc = acc + g[:, k * P * 4:(k + 1) * P * 4]
        hyp_k = jax.nn.sigmoid(acc) * mp
        for j in range(4):
            sel = (qq == 4 * pp + j).astype(f32)
            refs[j][k] = lax.dot_general(sel, hyp_k, (((0,), (1,)), ((), ())),
                                         preferred_element_type=f32)

    a0 = a0_ref[...]
    atb_ref[...] = lax.dot_general(bt_ref[...], a0,
                                   (((2,), (1,)), ((0,), (0,))),
                                   preferred_element_type=f32)
    y_ref[...] = y0_ref[...]
    u_ref[...] = u0_ref[...]
    d_ref[...] = d0_ref[...]


def _mega(c4, x3, wc1, bc1, wc2, bc2, wf1, bf1, wf2, bf2, mp,
          a0, bt, y0, u0, d0):
    full = lambda arr: pl.BlockSpec(arr.shape, lambda s: (0,) * arr.ndim)
    args = (c4, x3, wc1, bc1.reshape(1, -1), wc2, bc2.reshape(1, -1),
            wf1, bf1.reshape(1, -1), wf2, bf2.reshape(1, -1), mp,
            a0, bt, y0, u0, d0)
    return pl.pallas_call(
        _mega_kernel,
        grid=(K_IT + 1,),
        in_specs=[full(a) for a in args],
        out_specs=pl.BlockSpec((1, B, P, N_DIM),
                               lambda s: (jnp.clip(s - 1, 0, K_IT - 1), 0, 0, 0)),
        out_shape=jax.ShapeDtypeStruct((K_IT, B, P, N_DIM), jnp.float32),
        scratch_shapes=[
            pltpu.VMEM((P, B, N_DIM), jnp.float32),
            pltpu.VMEM((P, B, N_DIM), jnp.float32),
            pltpu.VMEM((P, B, N_DIM), jnp.float32),
            pltpu.VMEM((P, B, N_DIM), jnp.float32),
            pltpu.VMEM((B, P, P), jnp.float32),
            pltpu.VMEM((P, B), jnp.float32),
            pltpu.VMEM((K_IT, P, B), jnp.float32),
            pltpu.VMEM((K_IT, P, B), jnp.float32),
            pltpu.VMEM((K_IT, P, B), jnp.float32),
            pltpu.VMEM((K_IT, P, B), jnp.float32),
        ],
        compiler_params=pltpu.CompilerParams(
            dimension_semantics=("arbitrary",)),
    )(*args)


def kernel(b, A, W1, b1, W2, b2, W3, b3, Wc1, bc1, Wc2, bc2,
           Wf1, bf1, Wf2, bf2, max_param, edge_index):
    edge = edge_index.astype(jnp.int32)
    c4 = _sc_edge_counts(edge)

    x0 = b.reshape(B, P * M)
    x3 = _mlp3(x0, W1, b1, W2, b2, W3, b3)

    mp = max_param.reshape(1, 4)
    a0 = A[0]
    bt = jnp.transpose(b[..., 0], (1, 0, 2))

    ys = _mega(c4, x3, Wc1, bc1, Wc2, bc2, Wf1, bf1, Wf2, bf2, mp,
               a0, bt, jnp.asarray(_Y0), jnp.asarray(_U0), jnp.asarray(_D0))
    return ys[..., None]


# speedup vs baseline: 1.0751x; 1.0005x over previous
import functools

import jax
import jax.numpy as jnp
import numpy as np
from jax import lax
from jax.experimental import pallas as pl
from jax.experimental.pallas import tpu as pltpu
from jax.experimental.pallas import tpu_sc as plsc

B = 16
P = 64
M = 32
N_DIM = 256
H = 64
K_IT = 10
E = 512


def _threefry2x32(k0, k1, x0, x1):
    rot = (13, 15, 26, 6, 17, 29, 16, 24)
    k0 = np.uint32(k0)
    k1 = np.uint32(k1)
    ks = (k0, k1, np.uint32(k0 ^ k1 ^ np.uint32(0x1BD11BDA)))
    x0 = (x0 + ks[0]).astype(np.uint32)
    x1 = (x1 + ks[1]).astype(np.uint32)
    for i in range(5):
        for r in rot[(i % 2) * 4:(i % 2) * 4 + 4]:
            x0 = (x0 + x1).astype(np.uint32)
            x1 = ((x1 << np.uint32(r)) | (x1 >> np.uint32(32 - r))).astype(np.uint32)
            x1 = (x1 ^ x0).astype(np.uint32)
        x0 = (x0 + ks[(i + 1) % 3]).astype(np.uint32)
        x1 = (x1 + ks[(i + 2) % 3] + np.uint32(i + 1)).astype(np.uint32)
    return x0, x1


def _erfinv64(x):
    x = x.astype(np.float64)
    w = -np.log1p(-x * x)
    p_lo = np.full_like(w, 2.81022636e-08)
    wl = w - 2.5
    for c in (3.43273939e-07, -3.5233877e-06, -4.39150654e-06, 0.00021858087,
              -0.00125372503, -0.00417768164, 0.246640727, 1.50140941):
        p_lo = c + p_lo * wl
    ws = np.sqrt(np.maximum(w, 5.0)) - 3.0
    p_hi = np.full_like(w, -0.000200214257)
    for c in (0.000100950558, 0.00134934322, -0.00367342844, 0.00573950773,
              -0.0076224613, 0.00943887047, 1.00167406, 2.83297682):
        p_hi = c + p_hi * ws
    return np.where(w < 5.0, p_lo, p_hi) * x


def _init_state():
    n = B * P * N_DIM
    with np.errstate(over="ignore"):
        s1, s2 = _threefry2x32(0, 1, np.zeros(3, np.uint32),
                               np.arange(3, dtype=np.uint32))
        out = []
        for i in range(3):
            b1, b2 = _threefry2x32(s1[i], s2[i], np.zeros(n, np.uint32),
                                   np.arange(n, dtype=np.uint32))
            bits = (b1 ^ b2).astype(np.uint32)
            f = ((bits >> np.uint32(9)) | np.uint32(0x3F800000)).view(np.float32)
            f = f - np.float32(1.0)
            lo = np.float32(np.nextafter(np.float32(-1.0), np.float32(0.0)))
            u = np.maximum(lo, (f * (np.float32(1.0) - lo) + lo).astype(np.float32))
            v = (np.sqrt(2.0) * _erfinv64(u)).astype(np.float32)
            v = v.reshape(B, P, N_DIM)
            out.append(np.transpose(v, (1, 0, 2)) * np.float32(0.01))
    return out


_Y0, _U0, _D0 = _init_state()


def _leaky(x):
    return jnp.where(x >= 0, x, 0.01 * x)


def _sc_edge_body(edge_hbm, c_hbm, src_v, dst_v, cnt_v):
    cid = lax.axis_index("c")
    sid = lax.axis_index("s")

    @pl.when(cid == 0)
    def _():
        bb = sid
        pltpu.sync_copy(edge_hbm.at[bb, 0], src_v)
        pltpu.sync_copy(edge_hbm.at[bb, 1], dst_v)
        zeros16 = jnp.zeros((16,), jnp.float32)

        def zbody(i, carry):
            cnt_v[pl.ds(i * 16, 16)] = zeros16
            return carry

        lax.fori_loop(0, P * P // 16, zbody, 0)
        lanes = lax.iota(jnp.int32, 16)
        ones16 = jnp.ones((16,), jnp.float32)
        for ch in range(E // 16):
            s = src_v[pl.ds(ch * 16, 16)]
            d = dst_v[pl.ds(ch * 16, 16)]
            flat = d * P + s
            for l in range(16):
                plsc.addupdate_scatter(cnt_v, [flat], ones16,
                                       mask=lanes == l)
        pltpu.sync_copy(cnt_v, c_hbm.at[bb])


def _sc_edge_counts(edge):
    mesh = plsc.VectorSubcoreMesh(core_axis_name="c", subcore_axis_name="s")
    fn = functools.partial(
        pl.kernel,
        mesh=mesh,
        out_type=jax.ShapeDtypeStruct((B, P * P), jnp.float32),
        scratch_types=[
            pltpu.VMEM((E,), jnp.int32),
            pltpu.VMEM((E,), jnp.int32),
            pltpu.VMEM((P * P,), jnp.float32),
        ],
        compiler_params=pltpu.CompilerParams(needs_layout_passes=False),
    )(_sc_edge_body)
    return fn(edge)


_M1 = 8
_M2 = _M1 + 16
_M3 = _M2 + 32


def _mlp3_kernel(x0_ref, w1_ref, b1_ref, w2_ref, b2_ref, w3_ref, b3_ref,
                 o_ref, x1_s, x2_s, acc1_s, acc2_s):
    f32 = jnp.float32
    s = pl.program_id(0)

    @pl.when(s < _M1)
    def _layer1():
        o = jnp.dot(x0_ref[...], w1_ref[...], preferred_element_type=f32)
        x1_s[:, pl.ds(s * 512, 512)] = _leaky(o + b1_ref[...])

    @pl.when(jnp.logical_and(s >= _M1, s < _M2))
    def _layer2():
        t = s - _M1
        k2 = t % 2
        j2 = t // 2

        @pl.when(k2 == 0)
        def _():
            acc1_s[...] = jnp.zeros_like(acc1_s)

        acc1_s[...] += jnp.dot(x1_s[:, pl.ds(k2 * 2048, 2048)], w2_ref[...],
                               preferred_element_type=f32)

        @pl.when(k2 == 1)
        def _():
            x2_s[:, pl.ds(j2 * 1024, 1024)] = _leaky(acc1_s[...] + b2_ref[...])

    @pl.when(s >= _M2)
    def _layer3():
        t = s - _M2
        k3 = t % 4
        j3 = t // 4

        @pl.when(k3 == 0)
        def _():
            acc2_s[...] = jnp.zeros_like(acc2_s)

        acc2_s[...] += jnp.dot(x2_s[:, pl.ds(k3 * 2048, 2048)], w3_ref[...],
                               preferred_element_type=f32)

        @pl.when(k3 == 3)
        def _():
            o_ref[...] = acc2_s[...] + b3_ref[...]


def _mlp3(x0, w1, b1, w2, b2, w3, b3):
    def w1_map(s):
        return (0, jnp.clip(s, 0, _M1 - 1))

    def w2_map(s):
        t = jnp.clip(s - _M1, 0, 15)
        return (t % 2, t // 2)

    def b2_map(s):
        t = jnp.clip(s - _M1, 0, 15)
        return (0, t // 2)

    def w3_map(s):
        t = jnp.clip(s - _M2, 0, 31)
        return (t % 4, t // 4)

    def b3_map(s):
        t = jnp.clip(s - _M2, 0, 31)
        return (0, t // 4)

    return pl.pallas_call(
        _mlp3_kernel,
        grid=(_M3,),
        in_specs=[
            pl.BlockSpec((B, 2048), lambda s: (0, 0)),
            pl.BlockSpec((2048, 512), w1_map),
            pl.BlockSpec((1, 512), w1_map),
            pl.BlockSpec((2048, 1024), w2_map),
            pl.BlockSpec((1, 1024), b2_map),
            pl.BlockSpec((2048, 2048), w3_map),
            pl.BlockSpec((1, 2048), b3_map),
        ],
        out_specs=pl.BlockSpec((B, 2048), b3_map),
        out_shape=jax.ShapeDtypeStruct((B, 16384), jnp.float32),
        scratch_shapes=[
            pltpu.VMEM((B, 4096), jnp.float32),
            pltpu.VMEM((B, 8192), jnp.float32),
            pltpu.VMEM((B, 1024), jnp.float32),
            pltpu.VMEM((B, 2048), jnp.float32),
        ],
        compiler_params=pltpu.CompilerParams(
            dimension_semantics=("arbitrary",)),
    )(x0, w1, b1.reshape(1, -1), w2, b2.reshape(1, -1),
      w3, b3.reshape(1, -1))


def _mega_kernel(c_ref, x_ref, wc1_ref, bc1_ref, wc2_ref, bc2_ref,
                 wf1_ref, bf1_ref, wf2_ref, bf2_ref, mp_ref,
                 a0_ref, bt_ref, y0_ref, u0_ref, d0_ref, o_ref,
                 atb_ref, y_ref, u_ref, d_ref, lap_s, sn_s,
                 ha_ref, ht_ref, hr_ref, he_ref):
    f32 = jnp.float32
    step_id = pl.program_id(0)

    @pl.when(step_id == 0)
    def _prologue():
        _mega_prologue(c_ref, x_ref, wc1_ref, bc1_ref, wc2_ref, bc2_ref,
                       wf1_ref, bf1_ref, wf2_ref, bf2_ref, mp_ref,
                       a0_ref, bt_ref, y0_ref, u0_ref, d0_ref,
                       atb_ref, y_ref, u_ref, d_ref, lap_s, sn_s,
                       ha_ref, ht_ref, hr_ref, he_ref)

    @pl.when(step_id > 0)
    def _admm_step():
        k = step_id - 1
        a0 = a0_ref[...]
        sn = sn_s[...][:, :, None]
        al = jnp.reshape(ha_ref[pl.ds(k, 1)], (P, B))[:, :, None]
        ta = jnp.reshape(ht_ref[pl.ds(k, 1)], (P, B))[:, :, None]
        rh = jnp.reshape(hr_ref[pl.ds(k, 1)], (P, B))[:, :, None]
        et = jnp.reshape(he_ref[pl.ds(k, 1)], (P, B))[:, :, None]
        y = y_ref[...]
        ay = lax.dot_general(y, a0, (((2,), (2,)), ((0,), (0,))),
                             preferred_element_type=f32)
        atay = lax.dot_general(ay, a0, (((2,), (1,)), ((0,), (0,))),
                               preferred_element_type=f32)
        grad = (atay - atb_ref[...] + jnp.sign(y) * ta
                + u_ref[...] * sn + d_ref[...] * rh)
        y_next = y - al * grad
        for bb in range(B):
            yb = y_next[:, bb, :]
            db = jnp.dot(lap_s[bb], yb, preferred_element_type=f32)
            d_ref[:, bb, :] = db
            o_ref[0, bb] = yb
        u_ref[...] = u_ref[...] + d_ref[...] * et
        y_ref[...] = y_next


def _mega_prologue(c_ref, x_ref, wc1_ref, bc1_ref, wc2_ref, bc2_ref,
                   wf1_ref, bf1_ref, wf2_ref, bf2_ref, mp_ref,
                   a0_ref, bt_ref, y0_ref, u0_ref, d0_ref,
                   atb_ref, y_ref, u_ref, d_ref, lap_s, sn_s,
                   ha_ref, ht_ref, hr_ref, he_ref):
    f32 = jnp.float32
    c = jnp.reshape(c_ref[...], (B, P, P))
    ii = lax.broadcasted_iota(jnp.int32, (P, P), 0)
    jj = lax.broadcasted_iota(jnp.int32, (P, P), 1)
    eye = (ii == jj).astype(f32)
    ct = lax.dot_general(c, eye, (((1,), (0,)), ((), ())),
                         preferred_element_type=f32)
    deg_d = jnp.sum(c, axis=2)
    deg_s = jnp.sum(ct, axis=2)
    dinv = lax.rsqrt(deg_d + 1.0)
    adj = dinv[:, :, None] * dinv[:, None, :] * (c + eye[None])
    lap_s[...] = eye[None] * (deg_s + deg_d)[:, :, None] - c - ct
    sn_s[...] = lax.dot_general(eye, deg_s, (((1,), (1,)), ((), ())),
                                preferred_element_type=f32)

    x = jnp.reshape(x_ref[...], (B, P, 4 * H))
    xw = lax.dot_general(x, wc1_ref[...], (((2,), (0,)), ((), ())),
                         preferred_element_type=f32)
    h = lax.dot_general(adj, xw, (((2,), (1,)), ((0,), (0,))),
                        preferred_element_type=f32)
    h = _leaky(h + bc1_ref[...][None])
    hw = lax.dot_general(h, wc2_ref[...], (((2,), (0,)), ((), ())),
                         preferred_element_type=f32)
    h2 = lax.dot_general(adj, hw, (((2,), (1,)), ((0,), (0,))),
                         preferred_element_type=f32)
    h2 = _leaky(h2 + bc2_ref[...][None])
    pool = jnp.mean(h2, axis=1)
    f = _leaky(jnp.dot(pool, wf1_ref[...],
                       preferred_element_type=f32) + bf1_ref[...])
    g = jnp.dot(f, wf2_ref[...],
                preferred_element_type=f32) + bf2_ref[...]
    tq = lax.broadcasted_iota(jnp.int32, (4, P * 4), 0)
    tp = lax.broadcasted_iota(jnp.int32, (4, P * 4), 1)
    tilemat = (tp % 4 == tq).astype(f32)
    mp = jnp.dot(mp_ref[...], tilemat, preferred_element_type=f32)

    qq = lax.broadcasted_iota(jnp.int32, (P * 4, P), 0)
    pp = lax.broadcasted_iota(jnp.int32, (P * 4, P), 1)
    refs = (ha_ref, ht_ref, hr_ref, he_ref)
    acc = jnp.zeros((B, P * 4), f32)
    for k in range(K_IT):
        acc = acc + g[:, k * P * 4:(k + 1) * P * 4]
        hyp_k = jax.nn.sigmoid(acc) * mp
        for j in range(4):
            sel = (qq == 4 * pp + j).astype(f32)
            refs[j][k] = lax.dot_general(sel, hyp_k, (((0,), (1,)), ((), ())),
                                         preferred_element_type=f32)

    a0 = a0_ref[...]
    atb_ref[...] = lax.dot_general(bt_ref[...], a0,
                                   (((2,), (1,)), ((0,), (0,))),
                                   preferred_element_type=f32)
    y_ref[...] = y0_ref[...]
    u_ref[...] = u0_ref[...]
    d_ref[...] = d0_ref[...]


def _mega(c4, x3, wc1, bc1, wc2, bc2, wf1, bf1, wf2, bf2, mp,
          a0, bt, y0, u0, d0):
    full = lambda arr: pl.BlockSpec(arr.shape, lambda s: (0,) * arr.ndim)
    args = (c4, x3, wc1, bc1.reshape(1, -1), wc2, bc2.reshape(1, -1),
            wf1, bf1.reshape(1, -1), wf2, bf2.reshape(1, -1), mp,
            a0, bt, y0, u0, d0)
    return pl.pallas_call(
        _mega_kernel,
        grid=(K_IT + 1,),
        in_specs=[full(a) for a in args],
        out_specs=pl.BlockSpec((1, B, P, N_DIM),
                               lambda s: (jnp.clip(s - 1, 0, K_IT - 1), 0, 0, 0)),
        out_shape=jax.ShapeDtypeStruct((K_IT, B, P, N_DIM), jnp.float32),
        scratch_shapes=[
            pltpu.VMEM((P, B, N_DIM), jnp.float32),
            pltpu.VMEM((P, B, N_DIM), jnp.float32),
            pltpu.VMEM((P, B, N_DIM), jnp.float32),
            pltpu.VMEM((P, B, N_DIM), jnp.float32),
            pltpu.VMEM((B, P, P), jnp.float32),
            pltpu.VMEM((P, B), jnp.float32),
            pltpu.VMEM((K_IT, P, B), jnp.float32),
            pltpu.VMEM((K_IT, P, B), jnp.float32),
            pltpu.VMEM((K_IT, P, B), jnp.float32),
            pltpu.VMEM((K_IT, P, B), jnp.float32),
        ],
        compiler_params=pltpu.CompilerParams(
            dimension_semantics=("arbitrary",)),
    )(*args)


def kernel(b, A, W1, b1, W2, b2, W3, b3, Wc1, bc1, Wc2, bc2,
           Wf1, bf1, Wf2, bf2, max_param, edge_index):
    edge = edge_index.astype(jnp.int32)

    x0 = b.reshape(B, P * M)
    x3 = _mlp3(x0, W1, b1, W2, b2, W3, b3)
    c4 = _sc_edge_counts(edge)

    mp = max_param.reshape(1, 4)
    a0 = A[0]
    bt = jnp.transpose(b[..., 0], (1, 0, 2))

    ys = _mega(c4, x3, Wc1, bc1, Wc2, bc2, Wf1, bf1, Wf2, bf2, mp,
               a0, bt, jnp.asarray(_Y0), jnp.asarray(_U0), jnp.asarray(_D0))
    return ys[..., None]


# speedup vs baseline: 1.0756x; 1.0005x over previous
import functools

import jax
import jax.numpy as jnp
import numpy as np
from jax import lax
from jax.experimental import pallas as pl
from jax.experimental.pallas import tpu as pltpu
from jax.experimental.pallas import tpu_sc as plsc

B = 16
P = 64
M = 32
N_DIM = 256
H = 64
K_IT = 10
E = 512


def _threefry2x32(k0, k1, x0, x1):
    rot = (13, 15, 26, 6, 17, 29, 16, 24)
    k0 = np.uint32(k0)
    k1 = np.uint32(k1)
    ks = (k0, k1, np.uint32(k0 ^ k1 ^ np.uint32(0x1BD11BDA)))
    x0 = (x0 + ks[0]).astype(np.uint32)
    x1 = (x1 + ks[1]).astype(np.uint32)
    for i in range(5):
        for r in rot[(i % 2) * 4:(i % 2) * 4 + 4]:
            x0 = (x0 + x1).astype(np.uint32)
            x1 = ((x1 << np.uint32(r)) | (x1 >> np.uint32(32 - r))).astype(np.uint32)
            x1 = (x1 ^ x0).astype(np.uint32)
        x0 = (x0 + ks[(i + 1) % 3]).astype(np.uint32)
        x1 = (x1 + ks[(i + 2) % 3] + np.uint32(i + 1)).astype(np.uint32)
    return x0, x1


def _erfinv64(x):
    x = x.astype(np.float64)
    w = -np.log1p(-x * x)
    p_lo = np.full_like(w, 2.81022636e-08)
    wl = w - 2.5
    for c in (3.43273939e-07, -3.5233877e-06, -4.39150654e-06, 0.00021858087,
              -0.00125372503, -0.00417768164, 0.246640727, 1.50140941):
        p_lo = c + p_lo * wl
    ws = np.sqrt(np.maximum(w, 5.0)) - 3.0
    p_hi = np.full_like(w, -0.000200214257)
    for c in (0.000100950558, 0.00134934322, -0.00367342844, 0.00573950773,
              -0.0076224613, 0.00943887047, 1.00167406, 2.83297682):
        p_hi = c + p_hi * ws
    return np.where(w < 5.0, p_lo, p_hi) * x


def _init_state():
    n = B * P * N_DIM
    with np.errstate(over="ignore"):
        s1, s2 = _threefry2x32(0, 1, np.zeros(3, np.uint32),
                               np.arange(3, dtype=np.uint32))
        out = []
        for i in range(3):
            b1, b2 = _threefry2x32(s1[i], s2[i], np.zeros(n, np.uint32),
                                   np.arange(n, dtype=np.uint32))
            bits = (b1 ^ b2).astype(np.uint32)
            f = ((bits >> np.uint32(9)) | np.uint32(0x3F800000)).view(np.float32)
            f = f - np.float32(1.0)
            lo = np.float32(np.nextafter(np.float32(-1.0), np.float32(0.0)))
            u = np.maximum(lo, (f * (np.float32(1.0) - lo) + lo).astype(np.float32))
            v = (np.sqrt(2.0) * _erfinv64(u)).astype(np.float32)
            v = v.reshape(B, P, N_DIM)
            out.append(np.transpose(v, (1, 0, 2)) * np.float32(0.01))
    return out


_Y0, _U0, _D0 = _init_state()


def _leaky(x):
    return jnp.where(x >= 0, x, 0.01 * x)


def _sc_edge_body(edge_hbm, c_hbm, src_v, dst_v, cnt_v):
    cid = lax.axis_index("c")
    sid = lax.axis_index("s")

    @pl.when(cid == 0)
    def _():
        bb = sid
        pltpu.sync_copy(edge_hbm.at[bb, 0], src_v)
        pltpu.sync_copy(edge_hbm.at[bb, 1], dst_v)
        zeros16 = jnp.zeros((16,), jnp.float32)

        def zbody(i, carry):
            cnt_v[pl.ds(i * 16, 16)] = zeros16
            return carry

        lax.fori_loop(0, P * P // 16, zbody, 0)
        lanes = lax.iota(jnp.int32, 16)
        ones16 = jnp.ones((16,), jnp.float32)
        for ch in range(E // 16):
            s = src_v[pl.ds(ch * 16, 16)]
            d = dst_v[pl.ds(ch * 16, 16)]
            flat = d * P + s
            for l in range(16):
                plsc.addupdate_scatter(cnt_v, [flat], ones16,
                                       mask=lanes == l)
        pltpu.sync_copy(cnt_v, c_hbm.at[bb])


def _sc_edge_counts(edge):
    mesh = plsc.VectorSubcoreMesh(core_axis_name="c", subcore_axis_name="s")
    fn = functools.partial(
        pl.kernel,
        mesh=mesh,
        out_type=jax.ShapeDtypeStruct((B, P * P), jnp.float32),
        scratch_types=[
            pltpu.VMEM((E,), jnp.int32),
            pltpu.VMEM((E,), jnp.int32),
            pltpu.VMEM((P * P,), jnp.float32),
        ],
        compiler_params=pltpu.CompilerParams(needs_layout_passes=False),
    )(_sc_edge_body)
    return fn(edge)


_M1 = 8
_M2 = _M1 + 16
_M3 = _M2 + 32


def _mlp3_kernel(x0_ref, w1_ref, b1_ref, w2_ref, b2_ref, w3_ref, b3_ref,
                 o_ref, x1_s, x2_s, acc1_s, acc2_s):
    f32 = jnp.float32
    s = pl.program_id(0)

    @pl.when(s < _M1)
    def _layer1():
        o = jnp.dot(x0_ref[...], w1_ref[...], preferred_element_type=f32)
        x1_s[:, pl.ds(s * 512, 512)] = _leaky(o + b1_ref[...])

    @pl.when(jnp.logical_and(s >= _M1, s < _M2))
    def _layer2():
        t = s - _M1
        k2 = t % 2
        j2 = t // 2

        @pl.when(k2 == 0)
        def _():
            acc1_s[...] = jnp.zeros_like(acc1_s)

        acc1_s[...] += jnp.dot(x1_s[:, pl.ds(k2 * 2048, 2048)], w2_ref[...],
                               preferred_element_type=f32)

        @pl.when(k2 == 1)
        def _():
            x2_s[:, pl.ds(j2 * 1024, 1024)] = _leaky(acc1_s[...] + b2_ref[...])

    @pl.when(s >= _M2)
    def _layer3():
        t = s - _M2
        k3 = t % 4
        j3 = t // 4

        @pl.when(k3 == 0)
        def _():
            acc2_s[...] = jnp.zeros_like(acc2_s)

        acc2_s[...] += jnp.dot(x2_s[:, pl.ds(k3 * 2048, 2048)], w3_ref[...],
                               preferred_element_type=f32)

        @pl.when(k3 == 3)
        def _():
            o_ref[...] = acc2_s[...] + b3_ref[...]


def _mlp3(x0, w1, b1, w2, b2, w3, b3):
    def w1_map(s):
        return (0, jnp.clip(s, 0, _M1 - 1))

    def w2_map(s):
        t = jnp.clip(s - _M1, 0, 15)
        return (t % 2, t // 2)

    def b2_map(s):
        t = jnp.clip(s - _M1, 0, 15)
        return (0, t // 2)

    def w3_map(s):
        t = jnp.clip(s - _M2, 0, 31)
        return (t % 4, t // 4)

    def b3_map(s):
        t = jnp.clip(s - _M2, 0, 31)
        return (0, t // 4)

    return pl.pallas_call(
        _mlp3_kernel,
        grid=(_M3,),
        in_specs=[
            pl.BlockSpec((B, 2048), lambda s: (0, 0)),
            pl.BlockSpec((2048, 512), w1_map),
            pl.BlockSpec((1, 512), w1_map),
            pl.BlockSpec((2048, 1024), w2_map),
            pl.BlockSpec((1, 1024), b2_map),
            pl.BlockSpec((2048, 2048), w3_map),
            pl.BlockSpec((1, 2048), b3_map),
        ],
        out_specs=pl.BlockSpec((B, 2048), b3_map),
        out_shape=jax.ShapeDtypeStruct((B, 16384), jnp.float32),
        scratch_shapes=[
            pltpu.VMEM((B, 4096), jnp.float32),
            pltpu.VMEM((B, 8192), jnp.float32),
            pltpu.VMEM((B, 1024), jnp.float32),
            pltpu.VMEM((B, 2048), jnp.float32),
        ],
        compiler_params=pltpu.CompilerParams(
            dimension_semantics=("arbitrary",)),
    )(x0, w1, b1.reshape(1, -1), w2, b2.reshape(1, -1),
      w3, b3.reshape(1, -1))


def _mega_kernel(c_ref, x_ref, wc1_ref, bc1_ref, wc2_ref, bc2_ref,
                 wf1_ref, bf1_ref, wf2_ref, bf2_ref, mp_ref,
                 a0_ref, bt_ref, y0_ref, u0_ref, d0_ref, o_ref,
                 atb_ref, y_ref, u_ref, d_ref, lap_s, sn_s,
                 ha_ref, ht_ref, hr_ref, he_ref):
    f32 = jnp.float32
    step_id = pl.program_id(0)

    @pl.when(step_id == 0)
    def _prologue():
        _mega_prologue(c_ref, x_ref, wc1_ref, bc1_ref, wc2_ref, bc2_ref,
                       wf1_ref, bf1_ref, wf2_ref, bf2_ref, mp_ref,
                       a0_ref, bt_ref, y0_ref, u0_ref, d0_ref,
                       atb_ref, y_ref, u_ref, d_ref, lap_s, sn_s,
                       ha_ref, ht_ref, hr_ref, he_ref)

    @pl.when(step_id > 0)
    def _admm_step():
        k = step_id - 1
        a0 = a0_ref[...]
        sn = sn_s[...][:, :, None]
        al = jnp.reshape(ha_ref[pl.ds(k, 1)], (P, B))[:, :, None]
        ta = jnp.reshape(ht_ref[pl.ds(k, 1)], (P, B))[:, :, None]
        rh = jnp.reshape(hr_ref[pl.ds(k, 1)], (P, B))[:, :, None]
        et = jnp.reshape(he_ref[pl.ds(k, 1)], (P, B))[:, :, None]
        y = y_ref[...]
        ay = lax.dot_general(y, a0, (((2,), (2,)), ((0,), (0,))),
                             preferred_element_type=f32)
        atay = lax.dot_general(ay, a0, (((2,), (1,)), ((0,), (0,))),
                               preferred_element_type=f32)
        grad = (atay - atb_ref[...] + jnp.sign(y) * ta
                + u_ref[...] * sn + d_ref[...] * rh)
        y_next = y - al * grad
        for bb in range(B):
            yb = y_next[:, bb, :]
            db = jnp.dot(lap_s[bb], yb, preferred_element_type=f32)
            d_ref[:, bb, :] = db
            o_ref[0, bb] = yb
        u_ref[...] = u_ref[...] + d_ref[...] * et
        y_ref[...] = y_next


def _mega_prologue(c_ref, x_ref, wc1_ref, bc1_ref, wc2_ref, bc2_ref,
                   wf1_ref, bf1_ref, wf2_ref, bf2_ref, mp_ref,
                   a0_ref, bt_ref, y0_ref, u0_ref, d0_ref,
                   atb_ref, y_ref, u_ref, d_ref, lap_s, sn_s,
                   ha_ref, ht_ref, hr_ref, he_ref):
    f32 = jnp.float32
    c = jnp.reshape(c_ref[...], (B, P, P))
    ii = lax.broadcasted_iota(jnp.int32, (P, P), 0)
    jj = lax.broadcasted_iota(jnp.int32, (P, P), 1)
    eye = (ii == jj).astype(f32)
    ct = lax.dot_general(c, eye, (((1,), (0,)), ((), ())),
                         preferred_element_type=f32)
    deg_d = jnp.sum(c, axis=2)
    deg_s = jnp.sum(ct, axis=2)
    dinv = lax.rsqrt(deg_d + 1.0)
    adj = dinv[:, :, None] * dinv[:, None, :] * (c + eye[None])
    lap_s[...] = eye[None] * (deg_s + deg_d)[:, :, None] - c - ct
    sn_s[...] = lax.dot_general(eye, deg_s, (((1,), (1,)), ((), ())),
                                preferred_element_type=f32)

    x = jnp.reshape(x_ref[...], (B, P, 4 * H))
    xw = lax.dot_general(x, wc1_ref[...], (((2,), (0,)), ((), ())),
                         preferred_element_type=f32)
    h = lax.dot_general(adj, xw, (((2,), (1,)), ((0,), (0,))),
                        preferred_element_type=f32)
    h = _leaky(h + bc1_ref[...][None])
    hw = lax.dot_general(h, wc2_ref[...], (((2,), (0,)), ((), ())),
                         preferred_element_type=f32)
    h2 = lax.dot_general(adj, hw, (((2,), (1,)), ((0,), (0,))),
                         preferred_element_type=f32)
    h2 = _leaky(h2 + bc2_ref[...][None])
    pool = jnp.mean(h2, axis=1)
    f = _leaky(jnp.dot(pool, wf1_ref[...],
                       preferred_element_type=f32) + bf1_ref[...])
    g = jnp.dot(f, wf2_ref[...],
                preferred_element_type=f32) + bf2_ref[...]
    tq = lax.broadcasted_iota(jnp.int32, (4, P * 4), 0)
    tp = lax.broadcasted_iota(jnp.int32, (4, P * 4), 1)
    tilemat = (tp % 4 == tq).astype(f32)
    mp = jnp.dot(mp_ref[...], tilemat, preferred_element_type=f32)

    qq = lax.broadcasted_iota(jnp.int32, (P * 4, P), 0)
    pp = lax.broadcasted_iota(jnp.int32, (P * 4, P), 1)
    refs = (ha_ref, ht_ref, hr_ref, he_ref)
    acc = jnp.zeros((B, P * 4), f32)
    for k in range(K_IT):
        acc = acc + g[:, k * P * 4:(k + 1) * P * 4]
        hyp_k = jax.nn.sigmoid(acc) * mp
        for j in range(4):
            sel = (qq == 4 * pp + j).astype(f32)
            refs[j][k] = lax.dot_general(sel, hyp_k, (((0,), (1,)), ((), ())),
                                         preferred_element_type=f32)

    a0 = a0_ref[...]
    atb_ref[...] = lax.dot_general(bt_ref[...], a0,
                                   (((2,), (1,)), ((0,), (0,))),
                                   preferred_element_type=f32)
    y_ref[...] = y0_ref[...]
    u_ref[...] = u0_ref[...]
    d_ref[...] = d0_ref[...]


def _mega(c4, x3, wc1, bc1, wc2, bc2, wf1, bf1, wf2, bf2, mp,
          a0, bt, y0, u0, d0):
    full = lambda arr: pl.BlockSpec(arr.shape, lambda s: (0,) * arr.ndim)
    args = (c4, x3, wc1, bc1.reshape(1, -1), wc2, bc2.reshape(1, -1),
            wf1, bf1.reshape(1, -1), wf2, bf2.reshape(1, -1), mp,
            a0, bt, y0, u0, d0)
    return pl.pallas_call(
        _mega_kernel,
        grid=(K_IT + 1,),
        in_specs=[full(a) for a in args],
        out_specs=pl.BlockSpec((1, B, P, N_DIM),
                               lambda s: (jnp.clip(s - 1, 0, K_IT - 1), 0, 0, 0)),
        out_shape=jax.ShapeDtypeStruct((K_IT, B, P, N_DIM), jnp.float32),
        scratch_shapes=[
            pltpu.VMEM((P, B, N_DIM), jnp.float32),
            pltpu.VMEM((P, B, N_DIM), jnp.float32),
            pltpu.VMEM((P, B, N_DIM), jnp.float32),
            pltpu.VMEM((P, B, N_DIM), jnp.float32),
            pltpu.VMEM((B, P, P), jnp.float32),
            pltpu.VMEM((P, B), jnp.float32),
            pltpu.VMEM((K_IT, P, B), jnp.float32),
            pltpu.VMEM((K_IT, P, B), jnp.float32),
            pltpu.VMEM((K_IT, P, B), jnp.float32),
            pltpu.VMEM((K_IT, P, B), jnp.float32),
        ],
        compiler_params=pltpu.CompilerParams(
            dimension_semantics=("arbitrary",)),
    )(*args)


def kernel(b, A, W1, b1, W2, b2, W3, b3, Wc1, bc1, Wc2, bc2,
           Wf1, bf1, Wf2, bf2, max_param, edge_index):
    edge = edge_index.astype(jnp.int32)
    c4 = _sc_edge_counts(edge)

    x0 = b.reshape(B, P * M)
    x3 = _mlp3(x0, W1, b1, W2, b2, W3, b3)

    mp = max_param.reshape(1, 4)
    a0 = A[0]
    bt = jnp.transpose(b[..., 0], (1, 0, 2))

    ys = _mega(c4, x3, Wc1, bc1, Wc2, bc2, Wf1, bf1, Wf2, bf2, mp,
               a0, bt, jnp.asarray(_Y0), jnp.asarray(_U0), jnp.asarray(_D0))
    return ys[..., None]
